# Initial kernel scaffold; baseline (speedup 1.0000x reference)
import jax
import jax.numpy as jnp
from jax.experimental import pallas as pl


def kernel(encoder_output, opt, opt_len, qt_idx, opt_idx, W_embed, w_ih0, w_hh0, b_ih0, b_hh0, w_ih1, w_hh1, b_ih1, b_hh1, relevance):
    raise NotImplementedError("write your pallas kernel here")


# speedup vs baseline: 2.5057x; 2.5057x over previous
import functools

import jax
import jax.numpy as jnp
from jax.experimental import pallas as pl
from jax.experimental.pallas import tpu as pltpu

HID = 512
B, K, T = 64, 100, 20
N = B * K
N_C = 256
N_CHUNKS = N // N_C
REL_SUB = 240
REL_LANES = REL_SUB * 128


def _lstm_body(emb_ref, lens_ref, enc_ref, w0_ref, wh0_ref, b0_ref,
               w1_ref, wh1_ref, b1_ref, out_ref,
               h1_s, c1_s, h2_s, c2_s):
    zeros = jnp.zeros((N_C, HID), jnp.float32)
    h1_s[...] = zeros
    c1_s[...] = zeros
    h2_s[...] = zeros
    c2_s[...] = zeros
    lens = lens_ref[...]

    def step(t, _):
        x = emb_ref[t]
        mask = t < lens
        g = (jnp.dot(x, w0_ref[...], preferred_element_type=jnp.float32)
             + jnp.dot(h1_s[...], wh0_ref[...], preferred_element_type=jnp.float32)
             + b0_ref[...])
        i = jax.nn.sigmoid(g[:, 0:HID])
        f = jax.nn.sigmoid(g[:, HID:2 * HID])
        gg = jnp.tanh(g[:, 2 * HID:3 * HID])
        o = jax.nn.sigmoid(g[:, 3 * HID:4 * HID])
        c_new = f * c1_s[...] + i * gg
        h_new = o * jnp.tanh(c_new)
        c1_s[...] = jnp.where(mask, c_new, c1_s[...])
        y1 = jnp.where(mask, h_new, h1_s[...])
        h1_s[...] = y1
        g2 = (jnp.dot(y1, w1_ref[...], preferred_element_type=jnp.float32)
              + jnp.dot(h2_s[...], wh1_ref[...], preferred_element_type=jnp.float32)
              + b1_ref[...])
        i2 = jax.nn.sigmoid(g2[:, 0:HID])
        f2 = jax.nn.sigmoid(g2[:, HID:2 * HID])
        gg2 = jnp.tanh(g2[:, 2 * HID:3 * HID])
        o2 = jax.nn.sigmoid(g2[:, 3 * HID:4 * HID])
        c2_new = f2 * c2_s[...] + i2 * gg2
        h2_new = o2 * jnp.tanh(c2_new)
        c2_s[...] = jnp.where(mask, c2_new, c2_s[...])
        h2_s[...] = jnp.where(mask, h2_new, h2_s[...])
        return 0

    jax.lax.fori_loop(0, T, step, 0)
    s = jnp.sum(h2_s[...] * enc_ref[...], axis=1, keepdims=True)
    out_ref[...] = jnp.broadcast_to(s, (N_C, 128))


def _qt_body(qt_ref, opt_ref, rel_ref, out_ref):
    b = pl.program_id(0)
    iota8 = jax.lax.broadcasted_iota(jnp.int32, (8, 128), 0)
    lane = jax.lax.broadcasted_iota(jnp.int32, (1, 128), 1)
    acc = jnp.zeros((1, 128), jnp.float32)
    for k in range(K):
        idx = opt_ref[b, k]
        hi = idx >> 7
        lo = idx & 127
        chunk = rel_ref[0, pl.ds((hi >> 3) << 3, 8), :]
        row = jnp.sum(chunk * (iota8 == (hi & 7)).astype(jnp.float32),
                      axis=0, keepdims=True)
        val = row * (lane == lo).astype(jnp.float32)
        acc = acc + pltpu.roll(val, k - lo, axis=1)
    out_ref[0, 0, :] = acc[0, :]


@jax.jit
def kernel(encoder_output, opt, opt_len, qt_idx, opt_idx, W_embed,
           w_ih0, w_hh0, b_ih0, b_hh0, w_ih1, w_hh1, b_ih1, b_hh1,
           relevance):
    opt_t = opt.reshape(N, T).T
    emb = W_embed[opt_t] * (opt_t != 0)[..., None].astype(W_embed.dtype)
    lens_bc = jnp.broadcast_to(opt_len.reshape(N, 1), (N, HID)).astype(jnp.int32)
    enc_exp = jnp.broadcast_to(encoder_output[:, None, :], (B, K, HID)).reshape(N, HID)
    b0 = (b_ih0 + b_hh0).reshape(1, 4 * HID)
    b1 = (b_ih1 + b_hh1).reshape(1, 4 * HID)

    out2d = pl.pallas_call(
        _lstm_body,
        grid=(N_CHUNKS,),
        in_specs=[
            pl.BlockSpec((T, N_C, 300), lambda i: (0, i, 0)),
            pl.BlockSpec((N_C, HID), lambda i: (i, 0)),
            pl.BlockSpec((N_C, HID), lambda i: (i, 0)),
            pl.BlockSpec((300, 4 * HID), lambda i: (0, 0)),
            pl.BlockSpec((HID, 4 * HID), lambda i: (0, 0)),
            pl.BlockSpec((1, 4 * HID), lambda i: (0, 0)),
            pl.BlockSpec((HID, 4 * HID), lambda i: (0, 0)),
            pl.BlockSpec((HID, 4 * HID), lambda i: (0, 0)),
            pl.BlockSpec((1, 4 * HID), lambda i: (0, 0)),
        ],
        out_specs=pl.BlockSpec((N_C, 128), lambda i: (i, 0)),
        out_shape=jax.ShapeDtypeStruct((N, 128), jnp.float32),
        scratch_shapes=[pltpu.VMEM((N_C, HID), jnp.float32)] * 4,
        compiler_params=pltpu.CompilerParams(
            dimension_semantics=("parallel",),
            vmem_limit_bytes=100 * 1024 * 1024,
        ),
    )(emb, lens_bc, enc_exp, w_ih0.T, w_hh0.T, b0, w_ih1.T, w_hh1.T, b1)
    scores = out2d[:, 0].reshape(B, K)

    rel3d = jnp.pad(relevance.astype(jnp.float32),
                    ((0, 0), (0, REL_LANES - relevance.shape[1]))
                    ).reshape(relevance.shape[0], REL_SUB, 128)
    qt3d = pl.pallas_call(
        _qt_body,
        grid_spec=pltpu.PrefetchScalarGridSpec(
            num_scalar_prefetch=2,
            grid=(B,),
            in_specs=[
                pl.BlockSpec((1, REL_SUB, 128), lambda b, qt, op: (qt[b], 0, 0)),
            ],
            out_specs=pl.BlockSpec((1, 1, 128), lambda b, qt, op: (b, 0, 0)),
        ),
        out_shape=jax.ShapeDtypeStruct((B, 1, 128), jnp.float32),
    )(qt_idx.astype(jnp.int32), opt_idx.astype(jnp.int32), rel3d)
    qt_score = qt3d[:, 0, :K]
    return scores, qt_score


# speedup vs baseline: 2.6580x; 1.0608x over previous
import functools

import jax
import jax.numpy as jnp
from jax.experimental import pallas as pl
from jax.experimental.pallas import tpu as pltpu

HID = 512
B, K, T = 64, 100, 20
N = B * K
N_C = 256
N_CHUNKS = N // N_C
REL_SUB = 240
REL_LANES = REL_SUB * 128


def _lstm_body(opt_ref, w_emb_ref, lens_ref, enc_ref, w0_ref, wh0_ref, b0_ref,
               w1_ref, wh1_ref, b1_ref, out_ref,
               x_s, h1_s, c1_s, h2_s, c2_s):
    zeros = jnp.zeros((N_C, HID), jnp.float32)
    h1_s[...] = zeros
    c1_s[...] = zeros
    h2_s[...] = zeros
    c2_s[...] = zeros
    lens = lens_ref[...]

    iota8 = jax.lax.broadcasted_iota(jnp.int32, (8, 384), 0)

    def gbody(j, _):
        rows = []
        for q in range(8):
            idx = opt_ref[0, 0, j * 8 + q]
            base = pl.multiple_of((idx >> 3) << 3, 8)
            c = w_emb_ref[pl.ds(base, 8), :]
            rows.append(pltpu.roll(c, q - (idx & 7), axis=0))
        acc = rows[0]
        for q in range(1, 8):
            acc = jnp.where(iota8 == q, rows[q], acc)
        x_s[pl.ds(pl.multiple_of(j * 8, 8), 8), :] = acc
        return 0

    jax.lax.fori_loop(0, T * N_C // 8, gbody, 0)

    def sig(x):
        return 0.5 * jnp.tanh(0.5 * x) + 0.5

    def step(t, _):
        x = x_s[pl.ds(pl.multiple_of(t * N_C, 8), N_C), :].astype(jnp.bfloat16)
        mask = t < lens
        g = (jnp.dot(x, w0_ref[...], preferred_element_type=jnp.float32)
             + jnp.dot(h1_s[...].astype(jnp.bfloat16), wh0_ref[...],
                       preferred_element_type=jnp.float32)
             + b0_ref[...])
        i = sig(g[:, 0:HID])
        f = sig(g[:, HID:2 * HID])
        gg = jnp.tanh(g[:, 2 * HID:3 * HID])
        o = sig(g[:, 3 * HID:4 * HID])
        c_new = f * c1_s[...] + i * gg
        h_new = o * jnp.tanh(c_new)
        c1_s[...] = jnp.where(mask, c_new, c1_s[...])
        y1 = jnp.where(mask, h_new, h1_s[...])
        h1_s[...] = y1
        g2 = (jnp.dot(y1.astype(jnp.bfloat16), w1_ref[...],
                      preferred_element_type=jnp.float32)
              + jnp.dot(h2_s[...].astype(jnp.bfloat16), wh1_ref[...],
                        preferred_element_type=jnp.float32)
              + b1_ref[...])
        i2 = sig(g2[:, 0:HID])
        f2 = sig(g2[:, HID:2 * HID])
        gg2 = jnp.tanh(g2[:, 2 * HID:3 * HID])
        o2 = sig(g2[:, 3 * HID:4 * HID])
        c2_new = f2 * c2_s[...] + i2 * gg2
        h2_new = o2 * jnp.tanh(c2_new)
        c2_s[...] = jnp.where(mask, c2_new, c2_s[...])
        h2_s[...] = jnp.where(mask, h2_new, h2_s[...])
        return 0

    jax.lax.fori_loop(0, T, step, 0)
    s = jnp.sum(h2_s[...] * enc_ref[...], axis=1, keepdims=True)
    out_ref[...] = jnp.broadcast_to(s, (N_C, 128))


def _qt_body(qt_ref, opt_ref, rel_ref, out_ref):
    b = pl.program_id(0)
    iota8 = jax.lax.broadcasted_iota(jnp.int32, (8, 128), 0)
    lane = jax.lax.broadcasted_iota(jnp.int32, (1, 128), 1)
    acc = jnp.zeros((1, 128), jnp.float32)
    for k in range(K):
        idx = opt_ref[b, k]
        hi = idx >> 7
        lo = idx & 127
        chunk = rel_ref[0, pl.ds((hi >> 3) << 3, 8), :]
        row = jnp.sum(chunk * (iota8 == (hi & 7)).astype(jnp.float32),
                      axis=0, keepdims=True)
        val = row * (lane == lo).astype(jnp.float32)
        acc = acc + pltpu.roll(val, k - lo, axis=1)
    out_ref[0, 0, :] = acc[0, :]


@jax.jit
def kernel(encoder_output, opt, opt_len, qt_idx, opt_idx, W_embed,
           w_ih0, w_hh0, b_ih0, b_hh0, w_ih1, w_hh1, b_ih1, b_hh1,
           relevance):
    opt_tm = (opt.reshape(N, T).T.reshape(T, N_CHUNKS, N_C)
              .transpose(1, 0, 2).reshape(N_CHUNKS, 1, T * N_C).astype(jnp.int32))
    W_pad = jnp.pad(W_embed, ((0, 6), (0, 84)))
    w0p = jnp.pad(w_ih0.T, ((0, 84), (0, 0))).astype(jnp.bfloat16)
    lens_bc = jnp.broadcast_to(opt_len.reshape(N, 1), (N, HID)).astype(jnp.int32)
    enc_exp = jnp.broadcast_to(encoder_output[:, None, :], (B, K, HID)).reshape(N, HID)
    b0 = (b_ih0 + b_hh0).reshape(1, 4 * HID)
    b1 = (b_ih1 + b_hh1).reshape(1, 4 * HID)

    out2d = pl.pallas_call(
        _lstm_body,
        grid=(N_CHUNKS,),
        in_specs=[
            pl.BlockSpec((1, 1, T * N_C), lambda i: (i, 0, 0),
                         memory_space=pltpu.MemorySpace.SMEM),
            pl.BlockSpec(memory_space=pltpu.MemorySpace.VMEM),
            pl.BlockSpec((N_C, HID), lambda i: (i, 0)),
            pl.BlockSpec((N_C, HID), lambda i: (i, 0)),
            pl.BlockSpec((384, 4 * HID), lambda i: (0, 0)),
            pl.BlockSpec((HID, 4 * HID), lambda i: (0, 0)),
            pl.BlockSpec((1, 4 * HID), lambda i: (0, 0)),
            pl.BlockSpec((HID, 4 * HID), lambda i: (0, 0)),
            pl.BlockSpec((HID, 4 * HID), lambda i: (0, 0)),
            pl.BlockSpec((1, 4 * HID), lambda i: (0, 0)),
        ],
        out_specs=pl.BlockSpec((N_C, 128), lambda i: (i, 0)),
        out_shape=jax.ShapeDtypeStruct((N, 128), jnp.float32),
        scratch_shapes=[pltpu.VMEM((T * N_C, 384), jnp.float32)]
        + [pltpu.VMEM((N_C, HID), jnp.float32)] * 4,
        compiler_params=pltpu.CompilerParams(
            dimension_semantics=("parallel",),
            vmem_limit_bytes=100 * 1024 * 1024,
        ),
    )(opt_tm, W_pad, lens_bc, enc_exp,
      w0p, w_hh0.T.astype(jnp.bfloat16), b0,
      w_ih1.T.astype(jnp.bfloat16), w_hh1.T.astype(jnp.bfloat16), b1)
    scores = out2d[:, 0].reshape(B, K)

    rel3d = jnp.pad(relevance.astype(jnp.float32),
                    ((0, 0), (0, REL_LANES - relevance.shape[1]))
                    ).reshape(relevance.shape[0], REL_SUB, 128)
    qt3d = pl.pallas_call(
        _qt_body,
        grid_spec=pltpu.PrefetchScalarGridSpec(
            num_scalar_prefetch=2,
            grid=(B,),
            in_specs=[
                pl.BlockSpec((1, REL_SUB, 128), lambda b, qt, op: (qt[b], 0, 0)),
            ],
            out_specs=pl.BlockSpec((1, 1, 128), lambda b, qt, op: (b, 0, 0)),
        ),
        out_shape=jax.ShapeDtypeStruct((B, 1, 128), jnp.float32),
    )(qt_idx.astype(jnp.int32), opt_idx.astype(jnp.int32), rel3d)
    qt_score = qt3d[:, 0, :K]
    return scores, qt_score


# speedup vs baseline: 2.7116x; 1.0202x over previous
import functools

import jax
import jax.numpy as jnp
from jax.experimental import pallas as pl
from jax.experimental.pallas import tpu as pltpu

HID = 512
B, K, T = 64, 100, 20
N = B * K
N_C = 256
N_CHUNKS = N // N_C
REL_SUB = 240
REL_LANES = REL_SUB * 128


def _lstm_body(opt_ref, w_emb_ref, lens_ref, enc_ref, w0_ref, wh0_ref, b0_ref,
               w1_ref, wh1_ref, b1_ref, out_ref,
               x_s, h1_s, c1_s, h2_s, c2_s):
    zeros = jnp.zeros((N_C, HID), jnp.float32)
    h1_s[...] = zeros.astype(jnp.bfloat16)
    c1_s[...] = zeros
    h2_s[...] = zeros.astype(jnp.bfloat16)
    c2_s[...] = zeros
    lens = lens_ref[...]

    iota8 = jax.lax.broadcasted_iota(jnp.int32, (8, 384), 0)

    GU = 8

    def gbody(j, _):
        for jj in range(GU):
            g_idx = j * GU + jj
            rows = []
            for q in range(8):
                idx = opt_ref[0, 0, g_idx * 8 + q]
                base = pl.multiple_of((idx >> 3) << 3, 8)
                c = w_emb_ref[pl.ds(base, 8), :]
                rows.append(pltpu.roll(c, q - (idx & 7), axis=0))
            acc = rows[0]
            for q in range(1, 8):
                acc = jnp.where(iota8 == q, rows[q], acc)
            x_s[pl.ds(pl.multiple_of(g_idx * 8, 8), 8), :] = acc
        return 0

    jax.lax.fori_loop(0, T * N_C // (8 * GU), gbody, 0)

    def sig(x):
        return 0.5 * jnp.tanh(0.5 * x) + 0.5

    def step(t, _):
        x = x_s[pl.ds(pl.multiple_of(t * N_C, 8), N_C), :].astype(jnp.bfloat16)
        mask = t < lens
        g = (jnp.dot(x, w0_ref[...], preferred_element_type=jnp.float32)
             + jnp.dot(h1_s[...], wh0_ref[...],
                       preferred_element_type=jnp.float32)
             + b0_ref[...])
        i = sig(g[:, 0:HID])
        f = sig(g[:, HID:2 * HID])
        gg = jnp.tanh(g[:, 2 * HID:3 * HID])
        o = sig(g[:, 3 * HID:4 * HID])
        c_new = f * c1_s[...] + i * gg
        h_new = o * jnp.tanh(c_new)
        c1_s[...] = jnp.where(mask, c_new, c1_s[...])
        h1_s[...] = jnp.where(mask, h_new.astype(jnp.bfloat16), h1_s[...])
        y1 = h1_s[...]
        g2 = (jnp.dot(y1, w1_ref[...],
                      preferred_element_type=jnp.float32)
              + jnp.dot(h2_s[...], wh1_ref[...],
                        preferred_element_type=jnp.float32)
              + b1_ref[...])
        i2 = sig(g2[:, 0:HID])
        f2 = sig(g2[:, HID:2 * HID])
        gg2 = jnp.tanh(g2[:, 2 * HID:3 * HID])
        o2 = sig(g2[:, 3 * HID:4 * HID])
        c2_new = f2 * c2_s[...] + i2 * gg2
        h2_new = o2 * jnp.tanh(c2_new)
        c2_s[...] = jnp.where(mask, c2_new, c2_s[...])
        h2_s[...] = jnp.where(mask, h2_new.astype(jnp.bfloat16), h2_s[...])
        return 0

    jax.lax.fori_loop(0, T, step, 0)
    s = jnp.sum(h2_s[...].astype(jnp.float32) * enc_ref[...],
                axis=1, keepdims=True)
    out_ref[...] = jnp.broadcast_to(s, (N_C, 128))


def _qt_body(qt_ref, opt_ref, rel_ref, out_ref):
    b = pl.program_id(0)
    iota8 = jax.lax.broadcasted_iota(jnp.int32, (8, 128), 0)
    lane = jax.lax.broadcasted_iota(jnp.int32, (1, 128), 1)
    acc = jnp.zeros((1, 128), jnp.float32)
    for k in range(K):
        idx = opt_ref[b, k]
        hi = idx >> 7
        lo = idx & 127
        chunk = rel_ref[0, pl.ds((hi >> 3) << 3, 8), :]
        row = jnp.sum(chunk * (iota8 == (hi & 7)).astype(jnp.float32),
                      axis=0, keepdims=True)
        val = row * (lane == lo).astype(jnp.float32)
        acc = acc + pltpu.roll(val, k - lo, axis=1)
    out_ref[0, 0, :] = acc[0, :]


@jax.jit
def kernel(encoder_output, opt, opt_len, qt_idx, opt_idx, W_embed,
           w_ih0, w_hh0, b_ih0, b_hh0, w_ih1, w_hh1, b_ih1, b_hh1,
           relevance):
    opt_tm = (opt.reshape(N, T).T.reshape(T, N_CHUNKS, N_C)
              .transpose(1, 0, 2).reshape(N_CHUNKS, 1, T * N_C).astype(jnp.int32))
    W_pad = jnp.pad(W_embed, ((0, 6), (0, 84)))
    w0p = jnp.pad(w_ih0.T, ((0, 84), (0, 0))).astype(jnp.bfloat16)
    lens_bc = jnp.broadcast_to(opt_len.reshape(N, 1), (N, HID)).astype(jnp.int32)
    enc_exp = jnp.broadcast_to(encoder_output[:, None, :], (B, K, HID)).reshape(N, HID)
    b0 = (b_ih0 + b_hh0).reshape(1, 4 * HID)
    b1 = (b_ih1 + b_hh1).reshape(1, 4 * HID)

    out2d = pl.pallas_call(
        _lstm_body,
        grid=(N_CHUNKS,),
        in_specs=[
            pl.BlockSpec((1, 1, T * N_C), lambda i: (i, 0, 0),
                         memory_space=pltpu.MemorySpace.SMEM),
            pl.BlockSpec(memory_space=pltpu.MemorySpace.VMEM),
            pl.BlockSpec((N_C, HID), lambda i: (i, 0)),
            pl.BlockSpec((N_C, HID), lambda i: (i, 0)),
            pl.BlockSpec((384, 4 * HID), lambda i: (0, 0)),
            pl.BlockSpec((HID, 4 * HID), lambda i: (0, 0)),
            pl.BlockSpec((1, 4 * HID), lambda i: (0, 0)),
            pl.BlockSpec((HID, 4 * HID), lambda i: (0, 0)),
            pl.BlockSpec((HID, 4 * HID), lambda i: (0, 0)),
            pl.BlockSpec((1, 4 * HID), lambda i: (0, 0)),
        ],
        out_specs=pl.BlockSpec((N_C, 128), lambda i: (i, 0)),
        out_shape=jax.ShapeDtypeStruct((N, 128), jnp.float32),
        scratch_shapes=[pltpu.VMEM((T * N_C, 384), jnp.float32),
                        pltpu.VMEM((N_C, HID), jnp.bfloat16),
                        pltpu.VMEM((N_C, HID), jnp.float32),
                        pltpu.VMEM((N_C, HID), jnp.bfloat16),
                        pltpu.VMEM((N_C, HID), jnp.float32)],
        compiler_params=pltpu.CompilerParams(
            dimension_semantics=("arbitrary",),
            vmem_limit_bytes=100 * 1024 * 1024,
        ),
    )(opt_tm, W_pad, lens_bc, enc_exp,
      w0p, w_hh0.T.astype(jnp.bfloat16), b0,
      w_ih1.T.astype(jnp.bfloat16), w_hh1.T.astype(jnp.bfloat16), b1)
    scores = out2d[:, 0].reshape(B, K)

    rel3d = jnp.pad(relevance.astype(jnp.float32),
                    ((0, 0), (0, REL_LANES - relevance.shape[1]))
                    ).reshape(relevance.shape[0], REL_SUB, 128)
    qt3d = pl.pallas_call(
        _qt_body,
        grid_spec=pltpu.PrefetchScalarGridSpec(
            num_scalar_prefetch=2,
            grid=(B,),
            in_specs=[
                pl.BlockSpec((1, REL_SUB, 128), lambda b, qt, op: (qt[b], 0, 0)),
            ],
            out_specs=pl.BlockSpec((1, 1, 128), lambda b, qt, op: (b, 0, 0)),
        ),
        out_shape=jax.ShapeDtypeStruct((B, 1, 128), jnp.float32),
    )(qt_idx.astype(jnp.int32), opt_idx.astype(jnp.int32), rel3d)
    qt_score = qt3d[:, 0, :K]
    return scores, qt_score


# speedup vs baseline: 2.8415x; 1.0479x over previous
import functools

import jax
import jax.numpy as jnp
from jax.experimental import pallas as pl
from jax.experimental.pallas import tpu as pltpu

HID = 512
B, K, T = 64, 100, 20
N = B * K
N_C = 256
N_CHUNKS = N // N_C
REL_SUB = 240
REL_LANES = REL_SUB * 128


def _lstm_body(opt_ref, w_emb_ref, lens_ref, enc_ref, w0_ref, wh0_ref, b0_ref,
               w1_ref, wh1_ref, b1_ref, out_ref,
               xa_s, xb_s, h1_s, c1_s, h2_s, c2_s):
    zeros = jnp.zeros((N_C, HID), jnp.float32)
    h1_s[...] = zeros.astype(jnp.bfloat16)
    c1_s[...] = zeros
    h2_s[...] = zeros.astype(jnp.bfloat16)
    c2_s[...] = zeros
    lens = lens_ref[...]

    iota8 = jax.lax.broadcasted_iota(jnp.int32, (8, 384), 0)

    def gather_step(s, dest):
        for g in range(N_C // 8):
            rows = []
            for q in range(8):
                idx = opt_ref[0, 0, s * N_C + g * 8 + q]
                base = pl.multiple_of((idx >> 3) << 3, 8)
                c = w_emb_ref[pl.ds(base, 8), :]
                rows.append(pltpu.roll(c, q - (idx & 7), axis=0))
            acc = rows[0]
            for q in range(1, 8):
                acc = jnp.where(iota8 == q, rows[q], acc)
            dest[g * 8:(g + 1) * 8, :] = acc

    def sig(x):
        return 0.5 * jnp.tanh(0.5 * x) + 0.5

    def step(t, xsrc, xdst):
        gather_step(jnp.minimum(t + 1, T - 1), xdst)
        x = xsrc[...].astype(jnp.bfloat16)
        mask = t < lens
        g = (jnp.dot(x, w0_ref[...], preferred_element_type=jnp.float32)
             + jnp.dot(h1_s[...], wh0_ref[...],
                       preferred_element_type=jnp.float32)
             + b0_ref[...])
        i = sig(g[:, 0:HID])
        f = sig(g[:, HID:2 * HID])
        gg = jnp.tanh(g[:, 2 * HID:3 * HID])
        o = sig(g[:, 3 * HID:4 * HID])
        c_new = f * c1_s[...] + i * gg
        h_new = o * jnp.tanh(c_new)
        c1_s[...] = jnp.where(mask, c_new, c1_s[...])
        h1_s[...] = jnp.where(mask, h_new.astype(jnp.bfloat16), h1_s[...])
        y1 = h1_s[...]
        g2 = (jnp.dot(y1, w1_ref[...],
                      preferred_element_type=jnp.float32)
              + jnp.dot(h2_s[...], wh1_ref[...],
                        preferred_element_type=jnp.float32)
              + b1_ref[...])
        i2 = sig(g2[:, 0:HID])
        f2 = sig(g2[:, HID:2 * HID])
        gg2 = jnp.tanh(g2[:, 2 * HID:3 * HID])
        o2 = sig(g2[:, 3 * HID:4 * HID])
        c2_new = f2 * c2_s[...] + i2 * gg2
        h2_new = o2 * jnp.tanh(c2_new)
        c2_s[...] = jnp.where(mask, c2_new, c2_s[...])
        h2_s[...] = jnp.where(mask, h2_new.astype(jnp.bfloat16), h2_s[...])

    gather_step(0, xa_s)

    def pair(u, _):
        step(2 * u, xa_s, xb_s)
        step(2 * u + 1, xb_s, xa_s)
        return 0

    jax.lax.fori_loop(0, T // 2, pair, 0)
    s = jnp.sum(h2_s[...].astype(jnp.float32) * enc_ref[...],
                axis=1, keepdims=True)
    out_ref[...] = jnp.broadcast_to(s, (N_C, 128))


def _qt_body(qt_ref, opt_ref, rel_ref, out_ref):
    b = pl.program_id(0)
    iota8 = jax.lax.broadcasted_iota(jnp.int32, (8, 128), 0)
    lane = jax.lax.broadcasted_iota(jnp.int32, (1, 128), 1)
    acc = jnp.zeros((1, 128), jnp.float32)
    for k in range(K):
        idx = opt_ref[b, k]
        hi = idx >> 7
        lo = idx & 127
        chunk = rel_ref[0, pl.ds((hi >> 3) << 3, 8), :]
        row = jnp.sum(chunk * (iota8 == (hi & 7)).astype(jnp.float32),
                      axis=0, keepdims=True)
        val = row * (lane == lo).astype(jnp.float32)
        acc = acc + pltpu.roll(val, k - lo, axis=1)
    out_ref[0, 0, :] = acc[0, :]


@jax.jit
def kernel(encoder_output, opt, opt_len, qt_idx, opt_idx, W_embed,
           w_ih0, w_hh0, b_ih0, b_hh0, w_ih1, w_hh1, b_ih1, b_hh1,
           relevance):
    opt_tm = (opt.reshape(N, T).T.reshape(T, N_CHUNKS, N_C)
              .transpose(1, 0, 2).reshape(N_CHUNKS, 1, T * N_C).astype(jnp.int32))
    W_pad = jnp.pad(W_embed, ((0, 6), (0, 84)))
    w0p = jnp.pad(w_ih0.T, ((0, 84), (0, 0))).astype(jnp.bfloat16)
    lens_bc = jnp.broadcast_to(opt_len.reshape(N, 1), (N, HID)).astype(jnp.int32)
    enc_exp = jnp.broadcast_to(encoder_output[:, None, :], (B, K, HID)).reshape(N, HID)
    b0 = (b_ih0 + b_hh0).reshape(1, 4 * HID)
    b1 = (b_ih1 + b_hh1).reshape(1, 4 * HID)

    out2d = pl.pallas_call(
        _lstm_body,
        grid=(N_CHUNKS,),
        in_specs=[
            pl.BlockSpec((1, 1, T * N_C), lambda i: (i, 0, 0),
                         memory_space=pltpu.MemorySpace.SMEM),
            pl.BlockSpec(memory_space=pltpu.MemorySpace.VMEM),
            pl.BlockSpec((N_C, HID), lambda i: (i, 0)),
            pl.BlockSpec((N_C, HID), lambda i: (i, 0)),
            pl.BlockSpec((384, 4 * HID), lambda i: (0, 0)),
            pl.BlockSpec((HID, 4 * HID), lambda i: (0, 0)),
            pl.BlockSpec((1, 4 * HID), lambda i: (0, 0)),
            pl.BlockSpec((HID, 4 * HID), lambda i: (0, 0)),
            pl.BlockSpec((HID, 4 * HID), lambda i: (0, 0)),
            pl.BlockSpec((1, 4 * HID), lambda i: (0, 0)),
        ],
        out_specs=pl.BlockSpec((N_C, 128), lambda i: (i, 0)),
        out_shape=jax.ShapeDtypeStruct((N, 128), jnp.float32),
        scratch_shapes=[pltpu.VMEM((N_C, 384), jnp.float32),
                        pltpu.VMEM((N_C, 384), jnp.float32),
                        pltpu.VMEM((N_C, HID), jnp.bfloat16),
                        pltpu.VMEM((N_C, HID), jnp.float32),
                        pltpu.VMEM((N_C, HID), jnp.bfloat16),
                        pltpu.VMEM((N_C, HID), jnp.float32)],
        compiler_params=pltpu.CompilerParams(
            dimension_semantics=("arbitrary",),
            vmem_limit_bytes=100 * 1024 * 1024,
        ),
    )(opt_tm, W_pad, lens_bc, enc_exp,
      w0p, w_hh0.T.astype(jnp.bfloat16), b0,
      w_ih1.T.astype(jnp.bfloat16), w_hh1.T.astype(jnp.bfloat16), b1)
    scores = out2d[:, 0].reshape(B, K)

    rel3d = jnp.pad(relevance.astype(jnp.float32),
                    ((0, 0), (0, REL_LANES - relevance.shape[1]))
                    ).reshape(relevance.shape[0], REL_SUB, 128)
    qt3d = pl.pallas_call(
        _qt_body,
        grid_spec=pltpu.PrefetchScalarGridSpec(
            num_scalar_prefetch=2,
            grid=(B,),
            in_specs=[
                pl.BlockSpec((1, REL_SUB, 128), lambda b, qt, op: (qt[b], 0, 0)),
            ],
            out_specs=pl.BlockSpec((1, 1, 128), lambda b, qt, op: (b, 0, 0)),
        ),
        out_shape=jax.ShapeDtypeStruct((B, 1, 128), jnp.float32),
    )(qt_idx.astype(jnp.int32), opt_idx.astype(jnp.int32), rel3d)
    qt_score = qt3d[:, 0, :K]
    return scores, qt_score


# speedup vs baseline: 2.8600x; 1.0065x over previous
import functools

import jax
import jax.numpy as jnp
from jax.experimental import pallas as pl
from jax.experimental.pallas import tpu as pltpu

HID = 512
B, K, T = 64, 100, 20
N = B * K
N_C = 640
N_CHUNKS = N // N_C
REL_SUB = 240
REL_LANES = REL_SUB * 128


def _lstm_body(optb_ref, optr_ref, w_emb_ref, lens_ref, enc_ref, w0_ref, wh0_ref, b0_ref,
               w1_ref, wh1_ref, b1_ref, out_ref,
               xa_s, xb_s, h1_s, c1_s, h2_s, c2_s):
    zeros = jnp.zeros((N_C, HID), jnp.float32)
    h1_s[...] = zeros.astype(jnp.bfloat16)
    c1_s[...] = zeros
    h2_s[...] = zeros.astype(jnp.bfloat16)
    c2_s[...] = zeros
    lens = lens_ref[...]

    iota8 = jax.lax.broadcasted_iota(jnp.int32, (8, 384), 0)

    def gather_step(s, dest):
        for g in range(N_C // 8):
            rows = []
            for q in range(8):
                k = s * N_C + g * 8 + q
                base = pl.multiple_of(optb_ref[0, 0, k], 8)
                c = w_emb_ref[pl.ds(base, 8), :]
                rows.append(pltpu.roll(c, q - optr_ref[0, 0, k], axis=0))
            acc = rows[0]
            for q in range(1, 8):
                acc = jnp.where(iota8 == q, rows[q], acc)
            dest[g * 8:(g + 1) * 8, :] = acc

    def sig(x):
        return 0.5 * jnp.tanh(0.5 * x) + 0.5

    def step(t, xsrc, xdst):
        gather_step(jnp.minimum(t + 1, T - 1), xdst)
        x = xsrc[...].astype(jnp.bfloat16)
        mask = t < lens
        g = (jnp.dot(x, w0_ref[...], preferred_element_type=jnp.float32)
             + jnp.dot(h1_s[...], wh0_ref[...],
                       preferred_element_type=jnp.float32)
             + b0_ref[...])
        i = sig(g[:, 0:HID])
        f = sig(g[:, HID:2 * HID])
        gg = jnp.tanh(g[:, 2 * HID:3 * HID])
        o = sig(g[:, 3 * HID:4 * HID])
        c_new = f * c1_s[...] + i * gg
        h_new = o * jnp.tanh(c_new)
        c1_s[...] = jnp.where(mask, c_new, c1_s[...])
        h1_s[...] = jnp.where(mask, h_new.astype(jnp.bfloat16), h1_s[...])
        y1 = h1_s[...]
        g2 = (jnp.dot(y1, w1_ref[...],
                      preferred_element_type=jnp.float32)
              + jnp.dot(h2_s[...], wh1_ref[...],
                        preferred_element_type=jnp.float32)
              + b1_ref[...])
        i2 = sig(g2[:, 0:HID])
        f2 = sig(g2[:, HID:2 * HID])
        gg2 = jnp.tanh(g2[:, 2 * HID:3 * HID])
        o2 = sig(g2[:, 3 * HID:4 * HID])
        c2_new = f2 * c2_s[...] + i2 * gg2
        h2_new = o2 * jnp.tanh(c2_new)
        c2_s[...] = jnp.where(mask, c2_new, c2_s[...])
        h2_s[...] = jnp.where(mask, h2_new.astype(jnp.bfloat16), h2_s[...])

    gather_step(0, xa_s)

    def pair(u, _):
        step(2 * u, xa_s, xb_s)
        step(2 * u + 1, xb_s, xa_s)
        return 0

    jax.lax.fori_loop(0, T // 2, pair, 0)
    s = jnp.sum(h2_s[...].astype(jnp.float32) * enc_ref[...],
                axis=1, keepdims=True)
    out_ref[...] = jnp.broadcast_to(s, (N_C, 128))


def _qt_body(qt_ref, opt_ref, rel_ref, out_ref):
    b = pl.program_id(0)
    iota8 = jax.lax.broadcasted_iota(jnp.int32, (8, 128), 0)
    lane = jax.lax.broadcasted_iota(jnp.int32, (1, 128), 1)
    acc = jnp.zeros((1, 128), jnp.float32)
    for k in range(K):
        idx = opt_ref[b, k]
        hi = idx >> 7
        lo = idx & 127
        chunk = rel_ref[0, pl.ds((hi >> 3) << 3, 8), :]
        row = jnp.sum(chunk * (iota8 == (hi & 7)).astype(jnp.float32),
                      axis=0, keepdims=True)
        val = row * (lane == lo).astype(jnp.float32)
        acc = acc + pltpu.roll(val, k - lo, axis=1)
    out_ref[0, 0, :] = acc[0, :]


@jax.jit
def kernel(encoder_output, opt, opt_len, qt_idx, opt_idx, W_embed,
           w_ih0, w_hh0, b_ih0, b_hh0, w_ih1, w_hh1, b_ih1, b_hh1,
           relevance):
    opt_tm = (opt.reshape(N, T).T.reshape(T, N_CHUNKS, N_C)
              .transpose(1, 0, 2).reshape(N_CHUNKS, 1, T * N_C).astype(jnp.int32))
    W_pad = jnp.pad(W_embed, ((0, 6), (0, 84)))
    w0p = jnp.pad(w_ih0.T, ((0, 84), (0, 0))).astype(jnp.bfloat16)
    lens_bc = jnp.broadcast_to(opt_len.reshape(N, 1), (N, HID)).astype(jnp.int32)
    enc_exp = jnp.broadcast_to(encoder_output[:, None, :], (B, K, HID)).reshape(N, HID)
    b0 = (b_ih0 + b_hh0).reshape(1, 4 * HID)
    b1 = (b_ih1 + b_hh1).reshape(1, 4 * HID)

    out2d = pl.pallas_call(
        _lstm_body,
        grid=(N_CHUNKS,),
        in_specs=[
            pl.BlockSpec((1, 1, T * N_C), lambda i: (i, 0, 0),
                         memory_space=pltpu.MemorySpace.SMEM),
            pl.BlockSpec((1, 1, T * N_C), lambda i: (i, 0, 0),
                         memory_space=pltpu.MemorySpace.SMEM),
            pl.BlockSpec(memory_space=pltpu.MemorySpace.VMEM),
            pl.BlockSpec((N_C, HID), lambda i: (i, 0)),
            pl.BlockSpec((N_C, HID), lambda i: (i, 0)),
            pl.BlockSpec((384, 4 * HID), lambda i: (0, 0)),
            pl.BlockSpec((HID, 4 * HID), lambda i: (0, 0)),
            pl.BlockSpec((1, 4 * HID), lambda i: (0, 0)),
            pl.BlockSpec((HID, 4 * HID), lambda i: (0, 0)),
            pl.BlockSpec((HID, 4 * HID), lambda i: (0, 0)),
            pl.BlockSpec((1, 4 * HID), lambda i: (0, 0)),
        ],
        out_specs=pl.BlockSpec((N_C, 128), lambda i: (i, 0)),
        out_shape=jax.ShapeDtypeStruct((N, 128), jnp.float32),
        scratch_shapes=[pltpu.VMEM((N_C, 384), jnp.float32),
                        pltpu.VMEM((N_C, 384), jnp.float32),
                        pltpu.VMEM((N_C, HID), jnp.bfloat16),
                        pltpu.VMEM((N_C, HID), jnp.float32),
                        pltpu.VMEM((N_C, HID), jnp.bfloat16),
                        pltpu.VMEM((N_C, HID), jnp.float32)],
        compiler_params=pltpu.CompilerParams(
            dimension_semantics=("arbitrary",),
            vmem_limit_bytes=100 * 1024 * 1024,
        ),
    )(opt_tm & -8, opt_tm & 7, W_pad, lens_bc, enc_exp,
      w0p, w_hh0.T.astype(jnp.bfloat16), b0,
      w_ih1.T.astype(jnp.bfloat16), w_hh1.T.astype(jnp.bfloat16), b1)
    scores = out2d[:, 0].reshape(B, K)

    rel3d = jnp.pad(relevance.astype(jnp.float32),
                    ((0, 0), (0, REL_LANES - relevance.shape[1]))
                    ).reshape(relevance.shape[0], REL_SUB, 128)
    qt3d = pl.pallas_call(
        _qt_body,
        grid_spec=pltpu.PrefetchScalarGridSpec(
            num_scalar_prefetch=2,
            grid=(B,),
            in_specs=[
                pl.BlockSpec((1, REL_SUB, 128), lambda b, qt, op: (qt[b], 0, 0)),
            ],
            out_specs=pl.BlockSpec((1, 1, 128), lambda b, qt, op: (b, 0, 0)),
        ),
        out_shape=jax.ShapeDtypeStruct((B, 1, 128), jnp.float32),
    )(qt_idx.astype(jnp.int32), opt_idx.astype(jnp.int32), rel3d)
    qt_score = qt3d[:, 0, :K]
    return scores, qt_score
